# Initial kernel scaffold; baseline (speedup 1.0000x reference)
import jax
import jax.numpy as jnp
from jax.experimental import pallas as pl


def kernel(cls_logits, bbox_pred, priors):
    raise NotImplementedError("write your pallas kernel here")


# speedup vs baseline: 2.2685x; 2.2685x over previous
import functools

import jax
import jax.numpy as jnp
from jax import lax
from jax.experimental import pallas as pl

_NUM_CLASSES = 81
_CENTER_VAR = 0.1
_SIZE_VAR = 0.2
_CONF_TH = 0.01
_NMS_TH = 0.45
_PRE_NMS_K = 400
_MAX_PER_IMAGE = 100

_N = 8732
_NCHUNK = 4
_CHUNK = _N // _NCHUNK


def _prep_body(logits_ref, bbox_ref, priors_ref, scores_ref, boxes_ref):
    z = logits_ref[0]
    m = jnp.max(z, axis=-1, keepdims=True)
    e = jnp.exp(z - m)
    scores_ref[0] = e / jnp.sum(e, axis=-1, keepdims=True)
    loc = bbox_ref[0]
    pr = priors_ref[...]
    cxy = loc[:, :2] * _CENTER_VAR * pr[:, 2:] + pr[:, :2]
    wh = jnp.exp(loc[:, 2:] * _SIZE_VAR) * pr[:, 2:]
    half = wh * 0.5
    boxes_ref[0] = jnp.concatenate([cxy - half, cxy + half], axis=-1)


def _prep(cls_logits, bbox_pred, priors):
    B = cls_logits.shape[0]
    return pl.pallas_call(
        _prep_body,
        grid=(B,),
        in_specs=[
            pl.BlockSpec((1, _N, _NUM_CLASSES), lambda b: (b, 0, 0)),
            pl.BlockSpec((1, _N, 4), lambda b: (b, 0, 0)),
            pl.BlockSpec((_N, 4), lambda b: (0, 0)),
        ],
        out_specs=[
            pl.BlockSpec((1, _N, _NUM_CLASSES), lambda b: (b, 0, 0)),
            pl.BlockSpec((1, _N, 4), lambda b: (b, 0, 0)),
        ],
        out_shape=[
            jax.ShapeDtypeStruct((B, _N, _NUM_CLASSES), jnp.float32),
            jax.ShapeDtypeStruct((B, _N, 4), jnp.float32),
        ],
    )(cls_logits, bbox_pred, priors)


def _iou_one_to_many(box, boxes):
    lt = jnp.maximum(box[:2], boxes[:, :2])
    rb = jnp.minimum(box[2:], boxes[:, 2:])
    inter = jnp.clip(rb - lt, 0.0).prod(-1)
    a1 = jnp.clip(box[2] - box[0], 0.0) * jnp.clip(box[3] - box[1], 0.0)
    a2 = jnp.clip(boxes[:, 2] - boxes[:, 0], 0.0) * jnp.clip(boxes[:, 3] - boxes[:, 1], 0.0)
    return inter / (a1 + a2 - inter + 1e-9)


def _nms_image(scores_img, boxes_img):
    C = scores_img.shape[1]
    fg = scores_img[:, 1:]
    flat = fg.reshape(-1)
    top_scores, top_idx = jax.lax.top_k(flat, _PRE_NMS_K)
    box_idx = top_idx // (C - 1)
    labels = top_idx % (C - 1) + 1
    cand_boxes = boxes_img[box_idx]
    run0 = jnp.where(top_scores > _CONF_TH, top_scores, -jnp.inf)
    off_boxes = cand_boxes + labels[:, None].astype(cand_boxes.dtype) * 2.0

    def body(k, carry):
        run, sel_idx, sel_valid = carry
        i = jnp.argmax(run)
        valid_i = run[i] > -jnp.inf
        ious = _iou_one_to_many(off_boxes[i], off_boxes)
        run = jnp.where(ious > _NMS_TH, -jnp.inf, run)
        run = run.at[i].set(-jnp.inf)
        sel_idx = sel_idx.at[k].set(i)
        sel_valid = sel_valid.at[k].set(valid_i)
        return run, sel_idx, sel_valid

    run, idx, validb = lax.fori_loop(
        0, _MAX_PER_IMAGE, body,
        (run0, jnp.zeros(_MAX_PER_IMAGE, jnp.int32), jnp.zeros(_MAX_PER_IMAGE, jnp.bool_)))
    valid = validb.astype(boxes_img.dtype)
    out_boxes = cand_boxes[idx] * valid[:, None]
    out_scores = top_scores[idx] * valid
    out_labels = jnp.where(validb, labels[idx], 0)
    return out_boxes, out_scores, out_labels


def kernel(cls_logits, bbox_pred, priors):
    scores, boxes = _prep(cls_logits, bbox_pred, priors)
    return jax.vmap(_nms_image)(scores, boxes)


# speedup vs baseline: 16.9636x; 7.4780x over previous
import functools

import jax
import jax.numpy as jnp
from jax import lax
from jax.experimental import pallas as pl
from jax.experimental.pallas import tpu as pltpu
from jax.experimental.pallas import tpu_sc as plsc

_NUM_CLASSES = 81
_CP = 82
_CENTER_VAR = 0.1
_SIZE_VAR = 0.2
_CONF_TH = 0.01
_NMS_TH = 0.45
_PRE_NMS_K = 400
_MAX_PER_IMAGE = 100

_B = 8
_N = 8732
_TOTF = _N * _CP
_FPW = _TOTF // 4
_CHN = 16320
_NCH = (_FPW + 7 + _CHN - 1) // _CHN
_CLAMP = _TOTF - _CHN
_NB = 8192
_KLOC = 1024
_KD = 4 * (_KLOC + 16)
_NEG_INF = float("-inf")


def _prep_body(logits_ref, bbox_ref, priors_ref, scores_ref, boxes_ref):
    z = logits_ref[0]
    m = jnp.max(z, axis=-1, keepdims=True)
    e = jnp.exp(z - m)
    s = e / jnp.sum(e, axis=-1, keepdims=True)
    scores_ref[0] = jnp.concatenate(
        [s, jnp.zeros((_N, _CP - _NUM_CLASSES), jnp.float32)], axis=-1)
    loc = bbox_ref[0]
    pr = priors_ref[...]
    cxy = loc[:, :2] * _CENTER_VAR * pr[:, 2:] + pr[:, :2]
    wh = jnp.exp(loc[:, 2:] * _SIZE_VAR) * pr[:, 2:]
    half = wh * 0.5
    boxes_ref[0] = jnp.concatenate([cxy - half, cxy + half], axis=-1)


def _prep(cls_logits, bbox_pred, priors):
    return pl.pallas_call(
        _prep_body,
        grid=(_B,),
        in_specs=[
            pl.BlockSpec((1, _N, _NUM_CLASSES), lambda b: (b, 0, 0)),
            pl.BlockSpec((1, _N, 4), lambda b: (b, 0, 0)),
            pl.BlockSpec((_N, 4), lambda b: (0, 0)),
        ],
        out_specs=[
            pl.BlockSpec((1, _N, _CP), lambda b: (b, 0, 0)),
            pl.BlockSpec((1, _N, 4), lambda b: (b, 0, 0)),
        ],
        out_shape=[
            jax.ShapeDtypeStruct((_B, _N, _CP), jnp.float32),
            jax.ShapeDtypeStruct((_B, _N, 4), jnp.float32),
        ],
    )(cls_logits, bbox_pred, priors)


def _sc_body(scores_hbm, boxes_hbm, ob_hbm, os_hbm, ol_hbm,
             buf, hist, htmp, cs, ci, ds_, di_, run_, gidx,
             x1_, y1_, x2_, y2_, ar_, l2_, boxes_v, outb, outs, outl,
             comm, _hist_sh, _cs_sh, _ci_sh):
    core = lax.axis_index("c")
    sub = lax.axis_index("s")
    li = sub // 4
    q = sub % 4
    b = 4 * core + li
    leader_id = 4 * li
    is_leader = q == 0
    I16 = lax.iota(jnp.int32, 16)
    ones16 = jnp.ones((16,), jnp.int32)

    for t in range(5):
        comm[t] = jnp.int32(0)

    def zero_hist(j, _):
        hist[pl.ds(j * 16, 16)] = jnp.zeros((16,), jnp.int32)
        return 0
    lax.fori_loop(0, _NB // 16, zero_hist, 0)
    plsc.subcore_barrier()

    e0 = jnp.int32(q * _FPW)
    e1 = e0 + _FPW
    a0 = (e0 // 8) * 8

    def chunk_hist(i, p):
        off = jnp.minimum(a0 + i * _CHN, _CLAMP)
        pltpu.sync_copy(scores_hbm.at[pl.ds(b * _TOTF + off, _CHN)], buf)
        lo = jnp.maximum(p, off)
        hi = jnp.minimum(e1, off + _CHN)

        def vec_hist(j, _):
            vec = buf[pl.ds(j * 16, 16)]
            av = off + j * 16 + I16
            r = av % _CP
            m = (av >= lo) & (av < hi) & (r != 0) & (r != _NUM_CLASSES)
            bi = jnp.minimum((vec * float(_NB)).astype(jnp.int32), _NB - 1)
            plsc.addupdate_scatter(hist, [bi], ones16, mask=m)
            return 0
        lax.fori_loop(0, _CHN // 16, vec_hist, 0)
        return hi
    lax.fori_loop(0, _NCH, chunk_hist, e0)

    pltpu.sync_copy(hist, _hist_sh.at[pl.ds((li * 4 + q) * _NB, _NB)])
    plsc.subcore_barrier()

    @pl.when(is_leader)
    def _():
        for rgn in range(1, 4):
            pltpu.sync_copy(_hist_sh.at[pl.ds((li * 4 + rgn) * _NB, _NB)], htmp)

            def acc(j, _):
                sl = pl.ds(j * 16, 16)
                hist[sl] = hist[sl] + htmp[sl]
                return 0
            lax.fori_loop(0, _NB // 16, acc, 0)

        def scan(vi, carry):
            tot, bstar, found = carry
            v = (_NB // 16 - 1) - vi
            h = hist[pl.ds(v * 16, 16)]
            sfx = lax.rev(plsc.cumsum(lax.rev(h, (0,))), (0,))
            sx = tot + sfx
            nhit = jnp.sum((sx >= _PRE_NMS_K).astype(jnp.int32))
            anyhit = nhit > 0
            cand = v * 16 + nhit - 1
            bstar = jnp.where(found | ~anyhit, bstar, cand)
            found = found | anyhit
            return tot + jnp.sum(h), bstar, found
        _, bstar, _ = lax.fori_loop(
            0, _NB // 16, scan, (jnp.int32(0), jnp.int32(0), False))
        for w in range(4):
            plsc.fetch_and_add(comm.at[4], bstar, subcore_id=leader_id + w)

    plsc.subcore_barrier()
    bstar = comm[4]

    def chunk_cand(i, carry):
        p, cnt = carry
        off = jnp.minimum(a0 + i * _CHN, _CLAMP)
        pltpu.sync_copy(scores_hbm.at[pl.ds(b * _TOTF + off, _CHN)], buf)
        lo = jnp.maximum(p, off)
        hi = jnp.minimum(e1, off + _CHN)

        def vec_cand(j, cnt):
            vec = buf[pl.ds(j * 16, 16)]
            av = off + j * 16 + I16
            r = av % _CP
            m = (av >= lo) & (av < hi) & (r != 0) & (r != _NUM_CLASSES)
            bi = jnp.minimum((vec * float(_NB)).astype(jnp.int32), _NB - 1)
            m = m & (bi >= bstar)
            plsc.store_compressed(cs.at[pl.ds(cnt, 16)], vec, mask=m)
            plsc.store_compressed(ci.at[pl.ds(cnt, 16)], av, mask=m)
            return jnp.minimum(cnt + jnp.sum(m.astype(jnp.int32)), _KLOC)
        cnt = lax.fori_loop(0, _CHN // 16, vec_cand, cnt)
        return hi, cnt
    _, cnt = lax.fori_loop(0, _NCH, chunk_cand, (e0, jnp.int32(0)))

    pltpu.sync_copy(cs, _cs_sh.at[pl.ds((li * 4 + q) * (_KLOC + 16), _KLOC + 16)])
    pltpu.sync_copy(ci, _ci_sh.at[pl.ds((li * 4 + q) * (_KLOC + 16), _KLOC + 16)])
    plsc.fetch_and_add(comm.at[q], cnt, subcore_id=leader_id)
    plsc.subcore_barrier()

    @pl.when(is_leader)
    def _():
        def zero_dense(j, _):
            sl = pl.ds(j * 16, 16)
            ds_[sl] = jnp.zeros((16,), jnp.float32)
            di_[sl] = jnp.zeros((16,), jnp.int32)
            run_[sl] = jnp.full((16,), _NEG_INF)
            return 0
        lax.fori_loop(0, _KD // 16, zero_dense, 0)
        for t in range(8):
            outs[pl.ds(t * 16, 16)] = jnp.zeros((16,), jnp.float32)
            outl[pl.ds(t * 16, 16)] = jnp.zeros((16,), jnp.int32)
        for t in range(32):
            outb[pl.ds(t * 16, 16)] = jnp.zeros((16,), jnp.float32)

        tot = jnp.int32(0)
        for rgn in range(4):
            pltpu.sync_copy(_cs_sh.at[pl.ds((li * 4 + rgn) * (_KLOC + 16), _KLOC + 16)], cs)
            pltpu.sync_copy(_ci_sh.at[pl.ds((li * 4 + rgn) * (_KLOC + 16), _KLOC + 16)], ci)
            crgn = comm[rgn]

            def merge(j, off):
                vs = cs[pl.ds(j * 16, 16)]
                vi = ci[pl.ds(j * 16, 16)]
                m = (j * 16 + I16) < crgn
                plsc.store_compressed(ds_.at[pl.ds(off, 16)], vs, mask=m)
                plsc.store_compressed(di_.at[pl.ds(off, 16)], vi, mask=m)
                return off + jnp.sum(m.astype(jnp.int32))
            tot = lax.fori_loop(0, (crgn + 15) // 16, merge, tot)

        nv = (tot + 15) // 16

        def count_ge(t):
            def cb(j, acc):
                bt = plsc.bitcast(ds_[pl.ds(j * 16, 16)], jnp.int32)
                m = (bt >= t) & ((j * 16 + I16) < tot)
                return acc + jnp.sum(m.astype(jnp.int32))
            return lax.fori_loop(0, nv, cb, jnp.int32(0))

        def bis(_, lohi):
            lo, hi = lohi
            mid = (lo + hi + 1) // 2
            c = count_ge(mid)
            ok = c >= _PRE_NMS_K
            return jnp.where(ok, mid, lo), jnp.where(ok, hi, mid - 1)
        b400, _ = lax.fori_loop(
            0, 31, bis, (jnp.int32(0), jnp.int32(0x3F800000)))
        rneed = _PRE_NMS_K - count_ge(b400 + 1)

        def count_tie(ix):
            def cb(j, acc):
                bt = plsc.bitcast(ds_[pl.ds(j * 16, 16)], jnp.int32)
                vi = di_[pl.ds(j * 16, 16)]
                m = (bt == b400) & (vi <= ix) & ((j * 16 + I16) < tot)
                return acc + jnp.sum(m.astype(jnp.int32))
            return lax.fori_loop(0, nv, cb, jnp.int32(0))

        def bis2(_, lohi):
            lo, hi = lohi
            mid = (lo + hi) // 2
            ok = count_tie(mid) >= rneed
            return jnp.where(ok, lo, mid + 1), jnp.where(ok, mid, hi)
        _, istar = lax.fori_loop(
            0, 21, bis2, (jnp.int32(0), jnp.int32(_TOTF)))

        def prep_run(j, _):
            sl = pl.ds(j * 16, 16)
            vs = ds_[sl]
            vi = di_[sl]
            bt = plsc.bitcast(vs, jnp.int32)
            ok = (j * 16 + I16) < tot
            intop = ok & ((bt > b400) | ((bt == b400) & (vi <= istar)))
            run_[sl] = jnp.where(intop & (vs > _CONF_TH), vs, _NEG_INF)
            l2_[sl] = (vi % _CP).astype(jnp.float32) * 2.0
            gidx[sl] = vi // _CP
            return 0
        lax.fori_loop(0, nv, prep_run, 0)

        pltpu.sync_copy(boxes_hbm.at[pl.ds(b * _N * 4, _N * 4)], boxes_v)

        zero16 = jnp.zeros((16,), jnp.int32)
        def prep_geom(j, _):
            sl = pl.ds(j * 16, 16)
            base = gidx[sl] * 4
            bx1 = plsc.load_gather(boxes_v, [base])
            by1 = plsc.load_gather(boxes_v, [base + 1])
            bx2 = plsc.load_gather(boxes_v, [base + 2])
            by2 = plsc.load_gather(boxes_v, [base + 3])
            l2v = l2_[sl]
            x1_[sl] = bx1 + l2v
            y1_[sl] = by1 + l2v
            x2_[sl] = bx2 + l2v
            y2_[sl] = by2 + l2v
            ar_[sl] = (jnp.maximum(bx2 - bx1, 0.0)
                       * jnp.maximum(by2 - by1, 0.0))
            return 0
        lax.fori_loop(0, nv, prep_geom, 0)

        def nms(k, _):
            def pass1(j, mv):
                return jnp.maximum(mv, run_[pl.ds(j * 16, 16)])
            mv = lax.fori_loop(0, nv, pass1, jnp.full((16,), _NEG_INF))
            m = jnp.max(mv)
            valid = m > _NEG_INF

            def pass2(j, best):
                e = run_[pl.ds(j * 16, 16)] == m
                cand = jnp.where(e, j * 16 + I16, jnp.int32(2 ** 30))
                return jnp.minimum(best, cand)
            bestv = lax.fori_loop(
                0, nv, pass2, jnp.full((16,), jnp.int32(2 ** 30)))
            i = jnp.where(valid, jnp.min(bestv), jnp.int32(0))

            spl = jnp.full((16,), i)
            ws = plsc.load_gather(ds_, [spl])
            wl2 = plsc.load_gather(l2_, [spl])
            wx1 = plsc.load_gather(x1_, [spl])
            wy1 = plsc.load_gather(y1_, [spl])
            wx2 = plsc.load_gather(x2_, [spl])
            wy2 = plsc.load_gather(y2_, [spl])
            wa = plsc.load_gather(ar_, [spl])

            def supp(j, _):
                sl = pl.ds(j * 16, 16)
                ltx = jnp.maximum(wx1, x1_[sl])
                lty = jnp.maximum(wy1, y1_[sl])
                rbx = jnp.minimum(wx2, x2_[sl])
                rby = jnp.minimum(wy2, y2_[sl])
                inter = (jnp.maximum(rbx - ltx, 0.0)
                         * jnp.maximum(rby - lty, 0.0))
                iou = inter / (wa + ar_[sl] - inter + 1e-9)
                kill = (iou > _NMS_TH) | ((j * 16 + I16) == i)
                run_[sl] = jnp.where(kill, _NEG_INF, run_[sl])
                return 0
            lax.fori_loop(0, nv, supp, 0)

            vf = jnp.where(valid, jnp.float32(1.0), jnp.float32(0.0))
            vfv = jnp.full((16,), vf)
            kspl = jnp.full((16,), k)
            m0 = I16 == 0
            plsc.store_scatter(outs, [kspl], ws * vfv, mask=m0)
            labv = ((wl2 * 0.5).astype(jnp.int32)
                    * vfv.astype(jnp.int32))
            plsc.store_scatter(outl, [kspl], labv, mask=m0)
            k4 = kspl * 4
            plsc.store_scatter(outb, [k4], (wx1 - wl2) * vfv, mask=m0)
            plsc.store_scatter(outb, [k4 + 1], (wy1 - wl2) * vfv, mask=m0)
            plsc.store_scatter(outb, [k4 + 2], (wx2 - wl2) * vfv, mask=m0)
            plsc.store_scatter(outb, [k4 + 3], (wy2 - wl2) * vfv, mask=m0)
            return 0
        lax.fori_loop(0, _MAX_PER_IMAGE, nms, 0)

        pltpu.sync_copy(outb, ob_hbm.at[pl.ds(b * 512, 512)])
        pltpu.sync_copy(outs, os_hbm.at[pl.ds(b * 128, 128)])
        pltpu.sync_copy(outl, ol_hbm.at[pl.ds(b * 128, 128)])


def _sc_nms(scores_flat, boxes):
    mesh = plsc.VectorSubcoreMesh(core_axis_name="c", subcore_axis_name="s")

    kfn = pl.kernel(
        _sc_body,
        out_type=[
            jax.ShapeDtypeStruct((_B * 512,), jnp.float32),
            jax.ShapeDtypeStruct((_B * 128,), jnp.float32),
            jax.ShapeDtypeStruct((_B * 128,), jnp.int32),
        ],
        mesh=mesh,
        scratch_types=[
            pltpu.VMEM((_CHN,), jnp.float32),
            pltpu.VMEM((_NB,), jnp.int32),
            pltpu.VMEM((_NB,), jnp.int32),
            pltpu.VMEM((_KLOC + 16,), jnp.float32),
            pltpu.VMEM((_KLOC + 16,), jnp.int32),
            pltpu.VMEM((_KD,), jnp.float32),
            pltpu.VMEM((_KD,), jnp.int32),
            pltpu.VMEM((_KD,), jnp.float32),
            pltpu.VMEM((_KD,), jnp.int32),
            pltpu.VMEM((_KD,), jnp.float32),
            pltpu.VMEM((_KD,), jnp.float32),
            pltpu.VMEM((_KD,), jnp.float32),
            pltpu.VMEM((_KD,), jnp.float32),
            pltpu.VMEM((_KD,), jnp.float32),
            pltpu.VMEM((_KD,), jnp.float32),
            pltpu.VMEM((_N * 4,), jnp.float32),
            pltpu.VMEM((512,), jnp.float32),
            pltpu.VMEM((128,), jnp.float32),
            pltpu.VMEM((128,), jnp.int32),
            pltpu.SMEM((8,), jnp.int32),
            pltpu.VMEM_SHARED((16 * _NB,), jnp.int32),
            pltpu.VMEM_SHARED((16 * (_KLOC + 16),), jnp.float32),
            pltpu.VMEM_SHARED((16 * (_KLOC + 16),), jnp.int32),
        ],
        compiler_params=pltpu.CompilerParams(needs_layout_passes=False),
    )
    return kfn(scores_flat, boxes)


def kernel(cls_logits, bbox_pred, priors):
    scores, boxes = _prep(cls_logits, bbox_pred, priors)
    scores_flat = scores.reshape(_B * _TOTF)
    boxes_flat = boxes.reshape(_B * _N * 4)
    ob, osc, olb = _sc_nms(scores_flat, boxes_flat)
    ob = ob.reshape(_B, 128, 4)
    osc = osc.reshape(_B, 128)
    olb = olb.reshape(_B, 128)
    return (ob[:, :_MAX_PER_IMAGE, :], osc[:, :_MAX_PER_IMAGE],
            olb[:, :_MAX_PER_IMAGE])


# speedup vs baseline: 19.0442x; 1.1227x over previous
import functools

import jax
import jax.numpy as jnp
from jax import lax
from jax.experimental import pallas as pl
from jax.experimental.pallas import tpu as pltpu
from jax.experimental.pallas import tpu_sc as plsc

_NUM_CLASSES = 81
_CP = 128
_CENTER_VAR = 0.1
_SIZE_VAR = 0.2
_CONF_TH = 0.01
_NMS_TH = 0.45
_PRE_NMS_K = 400
_MAX_PER_IMAGE = 100

_B = 8
_N = 8732
_NP = 8736
_TOTF = _NP * _CP
_FPW = _TOTF // 4
_CHN = 16384
_NCH = (_FPW + _CHN - 1) // _CHN
_CLAMP = _TOTF - _CHN
_NB = 8192
_KLOC = 1024
_KD = 2080
_NEG_INF = float("-inf")


def _prep_body(logits_ref, bbox_ref, priors_ref, scores_ref, boxes_ref):
    z = logits_ref[0]
    m = jnp.max(z, axis=-1, keepdims=True)
    e = jnp.exp(z - m)
    s = e / jnp.sum(e, axis=-1, keepdims=True)
    sp = jnp.concatenate(
        [s, jnp.zeros((_N, _CP - _NUM_CLASSES), jnp.float32)], axis=-1)
    scores_ref[...] = jnp.concatenate(
        [sp, jnp.zeros((_NP - _N, _CP), jnp.float32)], axis=0)
    loc = bbox_ref[0]
    pr = priors_ref[...]
    cxy = loc[:, :2] * _CENTER_VAR * pr[:, 2:] + pr[:, :2]
    wh = jnp.exp(loc[:, 2:] * _SIZE_VAR) * pr[:, 2:]
    half = wh * 0.5
    boxes_ref[0] = jnp.concatenate([cxy - half, cxy + half], axis=-1)


def _prep(cls_logits, bbox_pred, priors):
    return pl.pallas_call(
        _prep_body,
        grid=(_B,),
        in_specs=[
            pl.BlockSpec((1, _N, _NUM_CLASSES), lambda b: (b, 0, 0)),
            pl.BlockSpec((1, _N, 4), lambda b: (b, 0, 0)),
            pl.BlockSpec((_N, 4), lambda b: (0, 0)),
        ],
        out_specs=[
            pl.BlockSpec((_NP, _CP), lambda b: (b, 0)),
            pl.BlockSpec((1, _N, 4), lambda b: (b, 0, 0)),
        ],
        out_shape=[
            jax.ShapeDtypeStruct((_B * _NP, _CP), jnp.float32),
            jax.ShapeDtypeStruct((_B, _N, 4), jnp.float32),
        ],
    )(cls_logits, bbox_pred, priors)


def _sc_body(scores_hbm, boxes_hbm, ob_hbm, os_hbm, ol_hbm,
             buf, hist, htmp, cs, ci, ds_, di_, run_, gidx,
             x1_, y1_, x2_, y2_, ar_, l2_, boxes_v, outb, outs, outl,
             comm, _hist_sh, _cs_sh, _ci_sh):
    core = lax.axis_index("c")
    sub = lax.axis_index("s")
    li = sub // 4
    q = sub % 4
    b = 4 * core + li
    leader_id = 4 * li
    is_leader = q == 0
    I16 = lax.iota(jnp.int32, 16)
    ones16 = jnp.ones((16,), jnp.int32)

    for t in range(5):
        comm[t] = jnp.int32(0)

    def zero_hist(j, _):
        hist[pl.ds(j * 16, 16)] = jnp.zeros((16,), jnp.int32)
        return 0
    lax.fori_loop(0, _NB // 16, zero_hist, 0)
    plsc.subcore_barrier()

    e0 = jnp.int32(q * _FPW)
    e1 = e0 + _FPW
    a0 = (e0 // 8) * 8

    def chunk_hist(i, p):
        off = jnp.minimum(a0 + i * _CHN, _CLAMP)
        pltpu.sync_copy(scores_hbm.at[pl.ds(b * _TOTF + off, _CHN)], buf)
        lo = jnp.maximum(p, off)
        hi = jnp.minimum(e1, off + _CHN)

        def vec_hist(j, _):
            vec = buf[pl.ds(j * 16, 16)]
            av = off + j * 16 + I16
            r = av & (_CP - 1)
            m = (av >= lo) & (av < hi) & (r >= 1) & (r <= _NUM_CLASSES - 1)
            bi = jnp.minimum((vec * float(_NB)).astype(jnp.int32), _NB - 1)
            plsc.addupdate_scatter(hist, [bi], ones16, mask=m)
            return 0
        lax.fori_loop(0, _CHN // 16, vec_hist, 0)
        return hi
    lax.fori_loop(0, _NCH, chunk_hist, e0)

    pltpu.sync_copy(hist, _hist_sh.at[pl.ds((li * 4 + q) * _NB, _NB)])
    plsc.subcore_barrier()

    @pl.when(is_leader)
    def _():
        for rgn in range(1, 4):
            pltpu.sync_copy(_hist_sh.at[pl.ds((li * 4 + rgn) * _NB, _NB)], htmp)

            def acc(j, _):
                sl = pl.ds(j * 16, 16)
                hist[sl] = hist[sl] + htmp[sl]
                return 0
            lax.fori_loop(0, _NB // 16, acc, 0)

        def scan(vi, carry):
            tot, bstar, found = carry
            v = (_NB // 16 - 1) - vi
            h = hist[pl.ds(v * 16, 16)]
            sfx = lax.rev(plsc.cumsum(lax.rev(h, (0,))), (0,))
            sx = tot + sfx
            nhit = jnp.sum((sx >= _PRE_NMS_K).astype(jnp.int32))
            anyhit = nhit > 0
            cand = v * 16 + nhit - 1
            bstar = jnp.where(found | ~anyhit, bstar, cand)
            found = found | anyhit
            return tot + jnp.sum(h), bstar, found
        _, bstar, _ = lax.fori_loop(
            0, _NB // 16, scan, (jnp.int32(0), jnp.int32(0), False))
        for w in range(4):
            plsc.fetch_and_add(comm.at[4], bstar, subcore_id=leader_id + w)

    plsc.subcore_barrier()
    bstar = comm[4]

    def chunk_cand(i, carry):
        p, cnt = carry
        off = jnp.minimum(a0 + i * _CHN, _CLAMP)
        pltpu.sync_copy(scores_hbm.at[pl.ds(b * _TOTF + off, _CHN)], buf)
        lo = jnp.maximum(p, off)
        hi = jnp.minimum(e1, off + _CHN)

        def vec_cand(j, cnt):
            vec = buf[pl.ds(j * 16, 16)]
            av = off + j * 16 + I16
            r = av & (_CP - 1)
            m = (av >= lo) & (av < hi) & (r >= 1) & (r <= _NUM_CLASSES - 1)
            bi = jnp.minimum((vec * float(_NB)).astype(jnp.int32), _NB - 1)
            m = m & (bi >= bstar)
            plsc.store_compressed(cs.at[pl.ds(cnt, 16)], vec, mask=m)
            plsc.store_compressed(ci.at[pl.ds(cnt, 16)], av, mask=m)
            return jnp.minimum(cnt + jnp.sum(m.astype(jnp.int32)), _KLOC)
        cnt = lax.fori_loop(0, _CHN // 16, vec_cand, cnt)
        return hi, cnt
    _, cnt = lax.fori_loop(0, _NCH, chunk_cand, (e0, jnp.int32(0)))

    pltpu.sync_copy(cs, _cs_sh.at[pl.ds((li * 4 + q) * (_KLOC + 16), _KLOC + 16)])
    pltpu.sync_copy(ci, _ci_sh.at[pl.ds((li * 4 + q) * (_KLOC + 16), _KLOC + 16)])
    plsc.fetch_and_add(comm.at[q], cnt, subcore_id=leader_id)
    plsc.subcore_barrier()

    @pl.when(is_leader)
    def _():
        def zero_dense(j, _):
            sl = pl.ds(j * 16, 16)
            ds_[sl] = jnp.zeros((16,), jnp.float32)
            di_[sl] = jnp.zeros((16,), jnp.int32)
            run_[sl] = jnp.full((16,), _NEG_INF)
            return 0
        lax.fori_loop(0, _KD // 16, zero_dense, 0)
        for t in range(8):
            outs[pl.ds(t * 16, 16)] = jnp.zeros((16,), jnp.float32)
            outl[pl.ds(t * 16, 16)] = jnp.zeros((16,), jnp.int32)
        for t in range(32):
            outb[pl.ds(t * 16, 16)] = jnp.zeros((16,), jnp.float32)

        tot = jnp.int32(0)
        for rgn in range(4):
            pltpu.sync_copy(_cs_sh.at[pl.ds((li * 4 + rgn) * (_KLOC + 16), _KLOC + 16)], cs)
            pltpu.sync_copy(_ci_sh.at[pl.ds((li * 4 + rgn) * (_KLOC + 16), _KLOC + 16)], ci)
            crgn = comm[rgn]

            def merge(j, off):
                vs = cs[pl.ds(j * 16, 16)]
                vi = ci[pl.ds(j * 16, 16)]
                m = (j * 16 + I16) < crgn
                plsc.store_compressed(ds_.at[pl.ds(off, 16)], vs, mask=m)
                plsc.store_compressed(di_.at[pl.ds(off, 16)], vi, mask=m)
                return off + jnp.sum(m.astype(jnp.int32))
            tot = lax.fori_loop(0, (crgn + 15) // 16, merge, tot)

        nv = (tot + 15) // 16

        def count_ge(t):
            def cb(j, acc):
                bt = plsc.bitcast(ds_[pl.ds(j * 16, 16)], jnp.int32)
                m = (bt >= t) & ((j * 16 + I16) < tot)
                return acc + jnp.sum(m.astype(jnp.int32))
            return lax.fori_loop(0, nv, cb, jnp.int32(0))

        def bis(_, lohi):
            lo, hi = lohi
            mid = (lo + hi + 1) // 2
            c = count_ge(mid)
            ok = c >= _PRE_NMS_K
            return jnp.where(ok, mid, lo), jnp.where(ok, hi, mid - 1)
        b400, _ = lax.fori_loop(
            0, 31, bis, (jnp.int32(0), jnp.int32(0x3F800000)))
        rneed = _PRE_NMS_K - count_ge(b400 + 1)

        def count_tie(ix):
            def cb(j, acc):
                bt = plsc.bitcast(ds_[pl.ds(j * 16, 16)], jnp.int32)
                vi = di_[pl.ds(j * 16, 16)]
                m = (bt == b400) & (vi <= ix) & ((j * 16 + I16) < tot)
                return acc + jnp.sum(m.astype(jnp.int32))
            return lax.fori_loop(0, nv, cb, jnp.int32(0))

        def bis2(_, lohi):
            lo, hi = lohi
            mid = (lo + hi) // 2
            ok = count_tie(mid) >= rneed
            return jnp.where(ok, lo, mid + 1), jnp.where(ok, mid, hi)
        _, istar = lax.fori_loop(
            0, 21, bis2, (jnp.int32(0), jnp.int32(_TOTF)))

        def prep_run(j, _):
            sl = pl.ds(j * 16, 16)
            vs = ds_[sl]
            vi = di_[sl]
            bt = plsc.bitcast(vs, jnp.int32)
            ok = (j * 16 + I16) < tot
            intop = ok & ((bt > b400) | ((bt == b400) & (vi <= istar)))
            run_[sl] = jnp.where(intop & (vs > _CONF_TH), vs, _NEG_INF)
            l2_[sl] = (vi & (_CP - 1)).astype(jnp.float32) * 2.0
            gidx[sl] = vi >> 7
            return 0
        lax.fori_loop(0, nv, prep_run, 0)

        pltpu.sync_copy(boxes_hbm.at[pl.ds(b * _N * 4, _N * 4)], boxes_v)

        zero16 = jnp.zeros((16,), jnp.int32)
        def prep_geom(j, _):
            sl = pl.ds(j * 16, 16)
            base = gidx[sl] * 4
            bx1 = plsc.load_gather(boxes_v, [base])
            by1 = plsc.load_gather(boxes_v, [base + 1])
            bx2 = plsc.load_gather(boxes_v, [base + 2])
            by2 = plsc.load_gather(boxes_v, [base + 3])
            l2v = l2_[sl]
            x1_[sl] = bx1 + l2v
            y1_[sl] = by1 + l2v
            x2_[sl] = bx2 + l2v
            y2_[sl] = by2 + l2v
            ar_[sl] = (jnp.maximum(bx2 - bx1, 0.0)
                       * jnp.maximum(by2 - by1, 0.0))
            return 0
        lax.fori_loop(0, nv, prep_geom, 0)

        def comp(j, off):
            sl = pl.ds(j * 16, 16)
            vr = run_[sl]
            vs = ds_[sl]
            vl = l2_[sl]
            v1 = x1_[sl]
            v2 = y1_[sl]
            v3 = x2_[sl]
            v4 = y2_[sl]
            va = ar_[sl]
            m = vr > _NEG_INF
            dsl = pl.ds(off, 16)
            plsc.store_compressed(run_.at[dsl], vr, mask=m)
            plsc.store_compressed(ds_.at[dsl], vs, mask=m)
            plsc.store_compressed(l2_.at[dsl], vl, mask=m)
            plsc.store_compressed(x1_.at[dsl], v1, mask=m)
            plsc.store_compressed(y1_.at[dsl], v2, mask=m)
            plsc.store_compressed(x2_.at[dsl], v3, mask=m)
            plsc.store_compressed(y2_.at[dsl], v4, mask=m)
            plsc.store_compressed(ar_.at[dsl], va, mask=m)
            return off + jnp.sum(m.astype(jnp.int32))
        nn = lax.fori_loop(0, nv, comp, jnp.int32(0))
        nv2 = (nn + 15) // 16

        def clr(j, _):
            sl = pl.ds(j * 16, 16)
            keep = (j * 16 + I16) < nn
            run_[sl] = jnp.where(keep, run_[sl], _NEG_INF)
            return 0
        lax.fori_loop(jnp.maximum(nv2 - 1, 0), nv2, clr, 0)

        def nms(k, _):
            def pass1(j, mv):
                return jnp.maximum(mv, run_[pl.ds(j * 16, 16)])
            mv = lax.fori_loop(0, nv2, pass1, jnp.full((16,), _NEG_INF))
            m = jnp.max(mv)
            valid = m > _NEG_INF

            def pass2(j, best):
                e = run_[pl.ds(j * 16, 16)] == m
                cand = jnp.where(e, j * 16 + I16, jnp.int32(2 ** 30))
                return jnp.minimum(best, cand)
            bestv = lax.fori_loop(
                0, nv2, pass2, jnp.full((16,), jnp.int32(2 ** 30)))
            i = jnp.where(valid, jnp.min(bestv), jnp.int32(0))

            spl = jnp.full((16,), i)
            ws = plsc.load_gather(ds_, [spl])
            wl2 = plsc.load_gather(l2_, [spl])
            wx1 = plsc.load_gather(x1_, [spl])
            wy1 = plsc.load_gather(y1_, [spl])
            wx2 = plsc.load_gather(x2_, [spl])
            wy2 = plsc.load_gather(y2_, [spl])
            wa = plsc.load_gather(ar_, [spl])

            def supp(j, _):
                sl = pl.ds(j * 16, 16)
                ltx = jnp.maximum(wx1, x1_[sl])
                lty = jnp.maximum(wy1, y1_[sl])
                rbx = jnp.minimum(wx2, x2_[sl])
                rby = jnp.minimum(wy2, y2_[sl])
                inter = (jnp.maximum(rbx - ltx, 0.0)
                         * jnp.maximum(rby - lty, 0.0))
                iou = inter / (wa + ar_[sl] - inter + 1e-9)
                kill = (iou > _NMS_TH) | ((j * 16 + I16) == i)
                run_[sl] = jnp.where(kill, _NEG_INF, run_[sl])
                return 0
            lax.fori_loop(0, nv2, supp, 0)

            vf = jnp.where(valid, jnp.float32(1.0), jnp.float32(0.0))
            vfv = jnp.full((16,), vf)
            kspl = jnp.full((16,), k)
            m0 = I16 == 0
            plsc.store_scatter(outs, [kspl], ws * vfv, mask=m0)
            labv = ((wl2 * 0.5).astype(jnp.int32)
                    * vfv.astype(jnp.int32))
            plsc.store_scatter(outl, [kspl], labv, mask=m0)
            k4 = kspl * 4
            plsc.store_scatter(outb, [k4], (wx1 - wl2) * vfv, mask=m0)
            plsc.store_scatter(outb, [k4 + 1], (wy1 - wl2) * vfv, mask=m0)
            plsc.store_scatter(outb, [k4 + 2], (wx2 - wl2) * vfv, mask=m0)
            plsc.store_scatter(outb, [k4 + 3], (wy2 - wl2) * vfv, mask=m0)
            return 0
        lax.fori_loop(0, _MAX_PER_IMAGE, nms, 0)

        pltpu.sync_copy(outb, ob_hbm.at[pl.ds(b * 512, 512)])
        pltpu.sync_copy(outs, os_hbm.at[pl.ds(b * 128, 128)])
        pltpu.sync_copy(outl, ol_hbm.at[pl.ds(b * 128, 128)])


def _sc_nms(scores_flat, boxes):
    mesh = plsc.VectorSubcoreMesh(core_axis_name="c", subcore_axis_name="s")

    kfn = pl.kernel(
        _sc_body,
        out_type=[
            jax.ShapeDtypeStruct((_B * 512,), jnp.float32),
            jax.ShapeDtypeStruct((_B * 128,), jnp.float32),
            jax.ShapeDtypeStruct((_B * 128,), jnp.int32),
        ],
        mesh=mesh,
        scratch_types=[
            pltpu.VMEM((_CHN,), jnp.float32),
            pltpu.VMEM((_NB,), jnp.int32),
            pltpu.VMEM((_NB,), jnp.int32),
            pltpu.VMEM((_KLOC + 16,), jnp.float32),
            pltpu.VMEM((_KLOC + 16,), jnp.int32),
            pltpu.VMEM((_KD,), jnp.float32),
            pltpu.VMEM((_KD,), jnp.int32),
            pltpu.VMEM((_KD,), jnp.float32),
            pltpu.VMEM((_KD,), jnp.int32),
            pltpu.VMEM((_KD,), jnp.float32),
            pltpu.VMEM((_KD,), jnp.float32),
            pltpu.VMEM((_KD,), jnp.float32),
            pltpu.VMEM((_KD,), jnp.float32),
            pltpu.VMEM((_KD,), jnp.float32),
            pltpu.VMEM((_KD,), jnp.float32),
            pltpu.VMEM((_N * 4,), jnp.float32),
            pltpu.VMEM((512,), jnp.float32),
            pltpu.VMEM((128,), jnp.float32),
            pltpu.VMEM((128,), jnp.int32),
            pltpu.SMEM((8,), jnp.int32),
            pltpu.VMEM_SHARED((16 * _NB,), jnp.int32),
            pltpu.VMEM_SHARED((16 * (_KLOC + 16),), jnp.float32),
            pltpu.VMEM_SHARED((16 * (_KLOC + 16),), jnp.int32),
        ],
        compiler_params=pltpu.CompilerParams(needs_layout_passes=False),
    )
    return kfn(scores_flat, boxes)


def kernel(cls_logits, bbox_pred, priors):
    scores, boxes = _prep(cls_logits, bbox_pred, priors)
    scores_flat = scores.reshape(_B * _TOTF)
    boxes_flat = boxes.reshape(_B * _N * 4)
    ob, osc, olb = _sc_nms(scores_flat, boxes_flat)
    ob = ob.reshape(_B, 128, 4)
    osc = osc.reshape(_B, 128)
    olb = olb.reshape(_B, 128)
    return (ob[:, :_MAX_PER_IMAGE, :], osc[:, :_MAX_PER_IMAGE],
            olb[:, :_MAX_PER_IMAGE])


# speedup vs baseline: 40.0116x; 2.1010x over previous
import functools

import jax
import jax.numpy as jnp
from jax import lax
from jax.experimental import pallas as pl
from jax.experimental.pallas import tpu as pltpu
from jax.experimental.pallas import tpu_sc as plsc

_NUM_CLASSES = 81
_CP = 128
_CENTER_VAR = 0.1
_SIZE_VAR = 0.2
_CONF_TH = 0.01
_NMS_TH = 0.45
_PRE_NMS_K = 400
_MAX_PER_IMAGE = 100

_B = 8
_N = 8732
_NP = 8736
_TOTF = _NP * _CP
_RPW = _NP // 4
_RC = 168
_NCHR = _RPW // _RC
_NB = 8192
_KLOC = 1024
_KD = 2080
_NEG_INF = float("-inf")


def _prep_body(logits_ref, bbox_ref, priors_ref, scores_ref, boxes_ref):
    z = logits_ref[0]
    m = jnp.max(z, axis=-1, keepdims=True)
    e = jnp.exp(z - m)
    s = e / jnp.sum(e, axis=-1, keepdims=True)
    sp = jnp.concatenate(
        [s, jnp.zeros((_N, _CP - _NUM_CLASSES), jnp.float32)], axis=-1)
    scores_ref[...] = jnp.concatenate(
        [sp, jnp.zeros((_NP - _N, _CP), jnp.float32)], axis=0)
    loc = bbox_ref[0]
    pr = priors_ref[...]
    cxy = loc[:, :2] * _CENTER_VAR * pr[:, 2:] + pr[:, :2]
    wh = jnp.exp(loc[:, 2:] * _SIZE_VAR) * pr[:, 2:]
    half = wh * 0.5
    boxes_ref[0] = jnp.concatenate([cxy - half, cxy + half], axis=-1)


def _prep(cls_logits, bbox_pred, priors):
    return pl.pallas_call(
        _prep_body,
        grid=(_B,),
        in_specs=[
            pl.BlockSpec((1, _N, _NUM_CLASSES), lambda b: (b, 0, 0)),
            pl.BlockSpec((1, _N, 4), lambda b: (b, 0, 0)),
            pl.BlockSpec((_N, 4), lambda b: (0, 0)),
        ],
        out_specs=[
            pl.BlockSpec((_NP, _CP), lambda b: (b, 0)),
            pl.BlockSpec((1, _N, 4), lambda b: (b, 0, 0)),
        ],
        out_shape=[
            jax.ShapeDtypeStruct((_B * _NP, _CP), jnp.float32),
            jax.ShapeDtypeStruct((_B, _N, 4), jnp.float32),
        ],
    )(cls_logits, bbox_pred, priors)


def _sc_body(scores_hbm, boxes_hbm, ob_hbm, os_hbm, ol_hbm,
             bufa, bufb, hist, htmp, cs, ci, ds_, di_, run_, gidx,
             x1_, y1_, x2_, y2_, ar_, l2_, boxes_v, outb, outs, outl,
             comm, sema, semb, _hist_sh, _cs_sh, _ci_sh):
    core = lax.axis_index("c")
    sub = lax.axis_index("s")
    li = sub // 4
    q = sub % 4
    b = 4 * core + li
    leader_id = 4 * li
    is_leader = q == 0
    I16 = lax.iota(jnp.int32, 16)
    ones16 = jnp.ones((16,), jnp.int32)

    for t in range(5):
        comm[t] = jnp.int32(0)

    def zero_hist(j, _):
        hist[pl.ds(j * 16, 16)] = jnp.zeros((16,), jnp.int32)
        return 0
    lax.fori_loop(0, _NB // 16, zero_hist, 0)
    plsc.subcore_barrier()

    rowbase0 = q * _RPW
    imgbase = b * _TOTF
    lane_not0 = I16 >= 1
    lane_is0 = I16 == 0

    def _src_at(ch):
        off = imgbase + (rowbase0 + ch * _RC) * _CP
        return scores_hbm.at[pl.ds(off, _RC * _CP)]

    def _lanemax(bufc, base):
        v0 = bufc[pl.ds(base, 16)]
        v1 = bufc[pl.ds(base + 16, 16)]
        v2 = bufc[pl.ds(base + 32, 16)]
        v3 = bufc[pl.ds(base + 48, 16)]
        v4 = bufc[pl.ds(base + 64, 16)]
        v5 = bufc[pl.ds(base + 80, 16)]
        v0 = jnp.where(lane_not0, v0, 0.0)
        v5 = jnp.where(lane_is0, v5, 0.0)
        return jnp.maximum(
            jnp.maximum(jnp.maximum(v0, v1), jnp.maximum(v2, v3)),
            jnp.maximum(v4, v5))

    cur = pltpu.async_copy(_src_at(0), bufa, sema)
    for ch in range(_NCHR):
        nxt = None
        if ch + 1 < _NCHR:
            nxt = pltpu.async_copy(
                _src_at(ch + 1),
                bufb if ch % 2 == 0 else bufa,
                semb if ch % 2 == 0 else sema)
        cur.wait()
        bufc = bufa if ch % 2 == 0 else bufb

        def rowa(rl, _, bufc=bufc):
            hm = _lanemax(bufc, rl * _CP)
            bi = jnp.minimum((hm * float(_NB)).astype(jnp.int32), _NB - 1)
            plsc.addupdate_scatter(hist, [bi], ones16)
            return 0
        lax.fori_loop(0, _RC, rowa, 0)
        cur = nxt

    pltpu.sync_copy(hist, _hist_sh.at[pl.ds((li * 4 + q) * _NB, _NB)])
    plsc.subcore_barrier()

    @pl.when(is_leader)
    def _():
        for rgn in range(1, 4):
            pltpu.sync_copy(_hist_sh.at[pl.ds((li * 4 + rgn) * _NB, _NB)], htmp)

            def acc(j, _):
                sl = pl.ds(j * 16, 16)
                hist[sl] = hist[sl] + htmp[sl]
                return 0
            lax.fori_loop(0, _NB // 16, acc, 0)

        def scan(vi, carry):
            tot, bstar, found = carry
            v = (_NB // 16 - 1) - vi
            h = hist[pl.ds(v * 16, 16)]
            sfx = lax.rev(plsc.cumsum(lax.rev(h, (0,))), (0,))
            sx = tot + sfx
            nhit = jnp.sum((sx >= _PRE_NMS_K).astype(jnp.int32))
            anyhit = nhit > 0
            cand = v * 16 + nhit - 1
            bstar = jnp.where(found | ~anyhit, bstar, cand)
            found = found | anyhit
            return tot + jnp.sum(h), bstar, found
        _, bstar, _ = lax.fori_loop(
            0, _NB // 16, scan, (jnp.int32(0), jnp.int32(0), False))
        for w in range(4):
            plsc.fetch_and_add(comm.at[4], bstar, subcore_id=leader_id + w)

    plsc.subcore_barrier()
    bstar = comm[4]

    tlo = bstar.astype(jnp.float32) * (1.0 / float(_NB))
    cnt = jnp.int32(0)
    cur = pltpu.async_copy(_src_at(0), bufa, sema)
    for ch in range(_NCHR):
        nxt = None
        if ch + 1 < _NCHR:
            nxt = pltpu.async_copy(
                _src_at(ch + 1),
                bufb if ch % 2 == 0 else bufa,
                semb if ch % 2 == 0 else sema)
        cur.wait()
        bufc = bufa if ch % 2 == 0 else bufb

        def rowb(rl, cnt, bufc=bufc, ch=ch):
            hm = _lanemax(bufc, rl * _CP)
            pc = plsc.all_reduce_population_count(hm >= tlo)

            def extract(c, bufc=bufc, rl=rl, ch=ch):
                base = rl * _CP
                av0 = (rowbase0 + ch * _RC + rl) * _CP + I16
                for k in range(6):
                    vk = bufc[pl.ds(base + 16 * k, 16)]
                    mk = vk >= tlo
                    if k == 0:
                        mk = mk & lane_not0
                    if k == 5:
                        mk = mk & lane_is0
                    plsc.store_compressed(cs.at[pl.ds(c, 16)], vk, mask=mk)
                    plsc.store_compressed(ci.at[pl.ds(c, 16)], av0 + 16 * k,
                                          mask=mk)
                    pck = plsc.all_reduce_population_count(mk)
                    c = jnp.minimum(c + pck[0], _KLOC)
                return c
            return lax.cond(pc[0] > 0, extract, lambda c: c, cnt)
        cnt = lax.fori_loop(0, _RC, rowb, cnt)
        cur = nxt

    pltpu.sync_copy(cs, _cs_sh.at[pl.ds((li * 4 + q) * (_KLOC + 16), _KLOC + 16)])
    pltpu.sync_copy(ci, _ci_sh.at[pl.ds((li * 4 + q) * (_KLOC + 16), _KLOC + 16)])
    plsc.fetch_and_add(comm.at[q], cnt, subcore_id=leader_id)
    plsc.subcore_barrier()

    @pl.when(is_leader)
    def _():
        def zero_dense(j, _):
            sl = pl.ds(j * 16, 16)
            ds_[sl] = jnp.zeros((16,), jnp.float32)
            di_[sl] = jnp.zeros((16,), jnp.int32)
            run_[sl] = jnp.full((16,), _NEG_INF)
            return 0
        lax.fori_loop(0, _KD // 16, zero_dense, 0)
        for t in range(8):
            outs[pl.ds(t * 16, 16)] = jnp.zeros((16,), jnp.float32)
            outl[pl.ds(t * 16, 16)] = jnp.zeros((16,), jnp.int32)
        for t in range(32):
            outb[pl.ds(t * 16, 16)] = jnp.zeros((16,), jnp.float32)

        tot = jnp.int32(0)
        for rgn in range(4):
            pltpu.sync_copy(_cs_sh.at[pl.ds((li * 4 + rgn) * (_KLOC + 16), _KLOC + 16)], cs)
            pltpu.sync_copy(_ci_sh.at[pl.ds((li * 4 + rgn) * (_KLOC + 16), _KLOC + 16)], ci)
            crgn = comm[rgn]

            def merge(j, off):
                vs = cs[pl.ds(j * 16, 16)]
                vi = ci[pl.ds(j * 16, 16)]
                m = (j * 16 + I16) < crgn
                plsc.store_compressed(ds_.at[pl.ds(off, 16)], vs, mask=m)
                plsc.store_compressed(di_.at[pl.ds(off, 16)], vi, mask=m)
                return off + jnp.sum(m.astype(jnp.int32))
            tot = lax.fori_loop(0, (crgn + 15) // 16, merge, tot)

        nv = (tot + 15) // 16

        def count_ge(t):
            def cb(j, acc):
                bt = plsc.bitcast(ds_[pl.ds(j * 16, 16)], jnp.int32)
                m = (bt >= t) & ((j * 16 + I16) < tot)
                return acc + jnp.sum(m.astype(jnp.int32))
            return lax.fori_loop(0, nv, cb, jnp.int32(0))

        def bis(_, lohi):
            lo, hi = lohi
            mid = (lo + hi + 1) // 2
            c = count_ge(mid)
            ok = c >= _PRE_NMS_K
            return jnp.where(ok, mid, lo), jnp.where(ok, hi, mid - 1)
        b400, _ = lax.fori_loop(
            0, 31, bis, (jnp.int32(0), jnp.int32(0x3F800000)))
        rneed = _PRE_NMS_K - count_ge(b400 + 1)

        def count_tie(ix):
            def cb(j, acc):
                bt = plsc.bitcast(ds_[pl.ds(j * 16, 16)], jnp.int32)
                vi = di_[pl.ds(j * 16, 16)]
                m = (bt == b400) & (vi <= ix) & ((j * 16 + I16) < tot)
                return acc + jnp.sum(m.astype(jnp.int32))
            return lax.fori_loop(0, nv, cb, jnp.int32(0))

        def bis2(_, lohi):
            lo, hi = lohi
            mid = (lo + hi) // 2
            ok = count_tie(mid) >= rneed
            return jnp.where(ok, lo, mid + 1), jnp.where(ok, mid, hi)
        _, istar = lax.fori_loop(
            0, 21, bis2, (jnp.int32(0), jnp.int32(_TOTF)))

        def prep_run(j, _):
            sl = pl.ds(j * 16, 16)
            vs = ds_[sl]
            vi = di_[sl]
            bt = plsc.bitcast(vs, jnp.int32)
            ok = (j * 16 + I16) < tot
            intop = ok & ((bt > b400) | ((bt == b400) & (vi <= istar)))
            run_[sl] = jnp.where(intop & (vs > _CONF_TH), vs, _NEG_INF)
            l2_[sl] = (vi & (_CP - 1)).astype(jnp.float32) * 2.0
            gidx[sl] = vi >> 7
            return 0
        lax.fori_loop(0, nv, prep_run, 0)

        pltpu.sync_copy(boxes_hbm.at[pl.ds(b * _N * 4, _N * 4)], boxes_v)

        zero16 = jnp.zeros((16,), jnp.int32)
        def prep_geom(j, _):
            sl = pl.ds(j * 16, 16)
            base = gidx[sl] * 4
            bx1 = plsc.load_gather(boxes_v, [base])
            by1 = plsc.load_gather(boxes_v, [base + 1])
            bx2 = plsc.load_gather(boxes_v, [base + 2])
            by2 = plsc.load_gather(boxes_v, [base + 3])
            l2v = l2_[sl]
            x1_[sl] = bx1 + l2v
            y1_[sl] = by1 + l2v
            x2_[sl] = bx2 + l2v
            y2_[sl] = by2 + l2v
            ar_[sl] = (jnp.maximum(bx2 - bx1, 0.0)
                       * jnp.maximum(by2 - by1, 0.0))
            return 0
        lax.fori_loop(0, nv, prep_geom, 0)

        def comp(j, off):
            sl = pl.ds(j * 16, 16)
            vr = run_[sl]
            vs = ds_[sl]
            vl = l2_[sl]
            v1 = x1_[sl]
            v2 = y1_[sl]
            v3 = x2_[sl]
            v4 = y2_[sl]
            va = ar_[sl]
            m = vr > _NEG_INF
            dsl = pl.ds(off, 16)
            plsc.store_compressed(run_.at[dsl], vr, mask=m)
            plsc.store_compressed(ds_.at[dsl], vs, mask=m)
            plsc.store_compressed(l2_.at[dsl], vl, mask=m)
            plsc.store_compressed(x1_.at[dsl], v1, mask=m)
            plsc.store_compressed(y1_.at[dsl], v2, mask=m)
            plsc.store_compressed(x2_.at[dsl], v3, mask=m)
            plsc.store_compressed(y2_.at[dsl], v4, mask=m)
            plsc.store_compressed(ar_.at[dsl], va, mask=m)
            return off + jnp.sum(m.astype(jnp.int32))
        nn = lax.fori_loop(0, nv, comp, jnp.int32(0))
        nv2 = (nn + 15) // 16

        def clr(j, _):
            sl = pl.ds(j * 16, 16)
            keep = (j * 16 + I16) < nn
            run_[sl] = jnp.where(keep, run_[sl], _NEG_INF)
            return 0
        lax.fori_loop(jnp.maximum(nv2 - 1, 0), nv2, clr, 0)

        def nms(k, _):
            def pass1(j, mv):
                return jnp.maximum(mv, run_[pl.ds(j * 16, 16)])
            mv = lax.fori_loop(0, nv2, pass1, jnp.full((16,), _NEG_INF))
            m = jnp.max(mv)
            valid = m > _NEG_INF

            def pass2(j, best):
                e = run_[pl.ds(j * 16, 16)] == m
                cand = jnp.where(e, j * 16 + I16, jnp.int32(2 ** 30))
                return jnp.minimum(best, cand)
            bestv = lax.fori_loop(
                0, nv2, pass2, jnp.full((16,), jnp.int32(2 ** 30)))
            i = jnp.where(valid, jnp.min(bestv), jnp.int32(0))

            spl = jnp.full((16,), i)
            ws = plsc.load_gather(ds_, [spl])
            wl2 = plsc.load_gather(l2_, [spl])
            wx1 = plsc.load_gather(x1_, [spl])
            wy1 = plsc.load_gather(y1_, [spl])
            wx2 = plsc.load_gather(x2_, [spl])
            wy2 = plsc.load_gather(y2_, [spl])
            wa = plsc.load_gather(ar_, [spl])

            def supp(j, _):
                sl = pl.ds(j * 16, 16)
                ltx = jnp.maximum(wx1, x1_[sl])
                lty = jnp.maximum(wy1, y1_[sl])
                rbx = jnp.minimum(wx2, x2_[sl])
                rby = jnp.minimum(wy2, y2_[sl])
                inter = (jnp.maximum(rbx - ltx, 0.0)
                         * jnp.maximum(rby - lty, 0.0))
                iou = inter / (wa + ar_[sl] - inter + 1e-9)
                kill = (iou > _NMS_TH) | ((j * 16 + I16) == i)
                run_[sl] = jnp.where(kill, _NEG_INF, run_[sl])
                return 0
            lax.fori_loop(0, nv2, supp, 0)

            vf = jnp.where(valid, jnp.float32(1.0), jnp.float32(0.0))
            vfv = jnp.full((16,), vf)
            kspl = jnp.full((16,), k)
            m0 = I16 == 0
            plsc.store_scatter(outs, [kspl], ws * vfv, mask=m0)
            labv = ((wl2 * 0.5).astype(jnp.int32)
                    * vfv.astype(jnp.int32))
            plsc.store_scatter(outl, [kspl], labv, mask=m0)
            k4 = kspl * 4
            plsc.store_scatter(outb, [k4], (wx1 - wl2) * vfv, mask=m0)
            plsc.store_scatter(outb, [k4 + 1], (wy1 - wl2) * vfv, mask=m0)
            plsc.store_scatter(outb, [k4 + 2], (wx2 - wl2) * vfv, mask=m0)
            plsc.store_scatter(outb, [k4 + 3], (wy2 - wl2) * vfv, mask=m0)
            return 0
        lax.fori_loop(0, _MAX_PER_IMAGE, nms, 0)

        pltpu.sync_copy(outb, ob_hbm.at[pl.ds(b * 512, 512)])
        pltpu.sync_copy(outs, os_hbm.at[pl.ds(b * 128, 128)])
        pltpu.sync_copy(outl, ol_hbm.at[pl.ds(b * 128, 128)])


def _sc_nms(scores_flat, boxes):
    mesh = plsc.VectorSubcoreMesh(core_axis_name="c", subcore_axis_name="s")

    kfn = pl.kernel(
        _sc_body,
        out_type=[
            jax.ShapeDtypeStruct((_B * 512,), jnp.float32),
            jax.ShapeDtypeStruct((_B * 128,), jnp.float32),
            jax.ShapeDtypeStruct((_B * 128,), jnp.int32),
        ],
        mesh=mesh,
        scratch_types=[
            pltpu.VMEM((_RC * _CP,), jnp.float32),
            pltpu.VMEM((_RC * _CP,), jnp.float32),
            pltpu.VMEM((_NB,), jnp.int32),
            pltpu.VMEM((_NB,), jnp.int32),
            pltpu.VMEM((_KLOC + 16,), jnp.float32),
            pltpu.VMEM((_KLOC + 16,), jnp.int32),
            pltpu.VMEM((_KD,), jnp.float32),
            pltpu.VMEM((_KD,), jnp.int32),
            pltpu.VMEM((_KD,), jnp.float32),
            pltpu.VMEM((_KD,), jnp.int32),
            pltpu.VMEM((_KD,), jnp.float32),
            pltpu.VMEM((_KD,), jnp.float32),
            pltpu.VMEM((_KD,), jnp.float32),
            pltpu.VMEM((_KD,), jnp.float32),
            pltpu.VMEM((_KD,), jnp.float32),
            pltpu.VMEM((_KD,), jnp.float32),
            pltpu.VMEM((_N * 4,), jnp.float32),
            pltpu.VMEM((512,), jnp.float32),
            pltpu.VMEM((128,), jnp.float32),
            pltpu.VMEM((128,), jnp.int32),
            pltpu.SMEM((8,), jnp.int32),
            pltpu.SemaphoreType.DMA,
            pltpu.SemaphoreType.DMA,
            pltpu.VMEM_SHARED((16 * _NB,), jnp.int32),
            pltpu.VMEM_SHARED((16 * (_KLOC + 16),), jnp.float32),
            pltpu.VMEM_SHARED((16 * (_KLOC + 16),), jnp.int32),
        ],
        compiler_params=pltpu.CompilerParams(needs_layout_passes=False),
    )
    return kfn(scores_flat, boxes)


def kernel(cls_logits, bbox_pred, priors):
    scores, boxes = _prep(cls_logits, bbox_pred, priors)
    scores_flat = scores.reshape(_B * _TOTF)
    boxes_flat = boxes.reshape(_B * _N * 4)
    ob, osc, olb = _sc_nms(scores_flat, boxes_flat)
    ob = ob.reshape(_B, 128, 4)
    osc = osc.reshape(_B, 128)
    olb = olb.reshape(_B, 128)
    return (ob[:, :_MAX_PER_IMAGE, :], osc[:, :_MAX_PER_IMAGE],
            olb[:, :_MAX_PER_IMAGE])
